# baseline (speedup 1.0000x reference)
import numpy as np

import jax
import jax.numpy as jnp
from jax import lax
from jax.experimental import pallas as pl
from jax.experimental.pallas import tpu as pltpu
from jax.experimental.pallas import tpu_sc as plsc

N = 10000
E = 320000
NW = 32
CHUNK = 128
NCH = E // CHUNK
FULL = NCH // NW
NTAIL = NCH - FULL * NW
TAIL_W0 = NW - NTAIL
NB = 6
LA = 3
ROWS_PS = N // 16

_f32 = jnp.float32
_SC_PARAMS = pltpu.CompilerParams(use_tc_tiling_on_sc=False, needs_layout_passes=False)
_MESH = dict(core_axis_name="c", subcore_axis_name="s", num_cores=2, num_subcores=16)


def _wid_base():
    c = lax.axis_index("c")
    s = lax.axis_index("s")
    wid = s * 2 + c
    base = wid * FULL + jnp.maximum(wid - TAIL_W0, 0)
    return wid, base


def _mm_body(x_ref, w_ref, o_ref):
    o_ref[...] = jnp.dot(x_ref[...], w_ref[...], preferred_element_type=_f32)


def _mm(x, w_cat):
    return pl.pallas_call(
        _mm_body,
        out_shape=jax.ShapeDtypeStruct((N, 32), _f32),
    )(x, w_cat)


def _deg_body(ei_hbm, part_hbm, idx_v, deg_v):
    wid, base = _wid_base()

    def zero_row(i, carry):
        deg_v[pl.ds(i * 16, 16)] = jnp.zeros((16,), _f32)
        return carry

    lax.fori_loop(0, N // 16, zero_row, 0)
    pltpu.sync_copy(ei_hbm.at[1, pl.ds(base, FULL + 1)], idx_v)

    ones = jnp.ones((16,), _f32)

    def hist_row(r):
        for k in range(CHUNK // 16):
            dvec = idx_v[r, pl.ds(k * 16, 16)]
            plsc.addupdate_scatter(deg_v, [dvec], ones)

    def row(r, carry):
        hist_row(r)
        return carry

    lax.fori_loop(0, FULL, row, 0)

    @pl.when(wid >= TAIL_W0)
    def _():
        hist_row(FULL)

    pltpu.sync_copy(deg_v, part_hbm.at[wid])


def _deg(ei3):
    return pl.kernel(
        _deg_body,
        out_type=jax.ShapeDtypeStruct((NW, N), _f32),
        mesh=plsc.VectorSubcoreMesh(**_MESH),
        scratch_types=[
            pltpu.VMEM((FULL + 1, CHUNK), jnp.int32),
            pltpu.VMEM((N,), _f32),
        ],
        compiler_params=_SC_PARAMS,
    )(ei3)


def _scale_body(part_ref, xw_ref, ones_ref, y_ref, dinv_ref):
    deg = (
        lax.dot_general(
            part_ref[...], ones_ref[...],
            (((0,), (0,)), ((), ())),
            preferred_element_type=_f32,
        )
        + 1.0
    )
    dinv = lax.rsqrt(deg)
    dinv_ref[...] = dinv
    y_ref[...] = dinv * xw_ref[...]


def _scale(partials, xw):
    ones = jnp.ones((NW, 1), _f32)
    return pl.pallas_call(
        _scale_body,
        out_shape=(
            jax.ShapeDtypeStruct((N, 32), _f32),
            jax.ShapeDtypeStruct((N, 1), _f32),
        ),
    )(partials, xw, ones)


def _edges_body(y_hbm, ei_hbm, acc_hbm, src_v, dst_v, rows, zero_v,
                y_sh, acc_sh, sem_g, sem_s):
    c = lax.axis_index("c")
    s = lax.axis_index("s")
    wid, base = _wid_base()

    def zero_row(i, carry):
        zero_v[i, pl.ds(0, 16)] = jnp.zeros((16,), _f32)
        zero_v[i, pl.ds(16, 16)] = jnp.zeros((16,), _f32)
        return carry

    lax.fori_loop(0, ROWS_PS, zero_row, 0)
    pltpu.sync_copy(zero_v, acc_sh.at[pl.ds(s * ROWS_PS, ROWS_PS)])
    pltpu.sync_copy(
        y_hbm.at[pl.ds(s * ROWS_PS, ROWS_PS)],
        y_sh.at[pl.ds(s * ROWS_PS, ROWS_PS)],
    )
    pltpu.sync_copy(ei_hbm.at[0, pl.ds(base, FULL + 1)], src_v)
    pltpu.sync_copy(ei_hbm.at[1, pl.ds(base, FULL + 1)], dst_v)
    plsc.subcore_barrier()

    def gather(i, b):
        pltpu.async_copy(y_sh.at[src_v.at[i]], rows.at[b], sem_g.at[b])

    def gather_wait(i, b):
        pltpu.make_async_copy(y_sh.at[src_v.at[i]], rows.at[b], sem_g.at[b]).wait()

    def scat(i, b):
        pltpu.async_copy(rows.at[b], acc_sh.at[dst_v.at[i]], sem_s.at[b], add=True)

    def scat_wait(i, b):
        pltpu.make_async_copy(rows.at[b], acc_sh.at[dst_v.at[i]], sem_s.at[b]).wait()

    for b in range(LA):
        gather(b, b)

    def group(g, carry):
        gbase = g * NB
        for b in range(NB):
            i = gbase + b
            gather_wait(i, b)
            scat(i, b)
            j = i + LA
            bj = (b + LA) % NB

            @pl.when(jnp.logical_and(j >= NB, j < FULL))
            def _():
                scat_wait(j - NB, bj)

            @pl.when(j < FULL)
            def _():
                gather(j, bj)
        return carry

    lax.fori_loop(0, FULL // NB, group, 0)
    for k in range(NB):
        scat_wait(FULL - NB + k, (FULL - NB + k) % NB)

    @pl.when(wid >= TAIL_W0)
    def _():
        pltpu.sync_copy(y_sh.at[src_v.at[FULL]], rows.at[0])
        pltpu.sync_copy(rows.at[0], acc_sh.at[dst_v.at[FULL]], add=True)

    plsc.subcore_barrier()
    pltpu.sync_copy(
        acc_sh.at[pl.ds(s * ROWS_PS, ROWS_PS)],
        acc_hbm.at[c, pl.ds(s * ROWS_PS, ROWS_PS)],
    )


def _edges(y, ei3):
    return pl.kernel(
        _edges_body,
        out_type=jax.ShapeDtypeStruct((2, N, 32), _f32),
        mesh=plsc.VectorSubcoreMesh(**_MESH),
        scratch_types=[
            pltpu.VMEM((FULL + 1, CHUNK), jnp.int32),
            pltpu.VMEM((FULL + 1, CHUNK), jnp.int32),
            pltpu.VMEM((NB, CHUNK, 32), _f32),
            pltpu.VMEM((ROWS_PS, 32), _f32),
            pltpu.VMEM_SHARED((N, 32), _f32),
            pltpu.VMEM_SHARED((N, 32), _f32),
            pltpu.SemaphoreType.DMA((NB,)),
            pltpu.SemaphoreType.DMA((NB,)),
        ],
        compiler_params=_SC_PARAMS,
    )(y, ei3)


ROWS_F = 312
ROWS_F_LAST = N - ROWS_F * (NW - 1)


def _finish_body(acc_hbm, y_hbm, dinv_hbm, b_hbm, eps_hbm, z_hbm, mu_hbm,
                 lv_hbm, a0_v, a1_v, y_v, dinv_v, eps_v, b_v, z_v, mu_v, lv_v):
    wid, _ = _wid_base()
    base = wid * ROWS_F
    pltpu.sync_copy(b_hbm, b_v)
    bm = b_v[pl.ds(0, 16)]
    bl = b_v[pl.ds(16, 16)]

    def run(nrows):
        pltpu.sync_copy(acc_hbm.at[0, pl.ds(base, nrows)], a0_v.at[pl.ds(0, nrows)])
        pltpu.sync_copy(acc_hbm.at[1, pl.ds(base, nrows)], a1_v.at[pl.ds(0, nrows)])
        pltpu.sync_copy(y_hbm.at[pl.ds(base, nrows)], y_v.at[pl.ds(0, nrows)])
        pltpu.sync_copy(dinv_hbm.at[pl.ds(base, nrows)], dinv_v.at[pl.ds(0, nrows)])
        pltpu.sync_copy(eps_hbm.at[pl.ds(base, nrows)], eps_v.at[pl.ds(0, nrows)])

        def row(r, carry):
            d16 = plsc.load_gather(dinv_v, [jnp.full((16,), 0, jnp.int32) + r])
            am = a0_v[r, pl.ds(0, 16)] + a1_v[r, pl.ds(0, 16)] + y_v[r, pl.ds(0, 16)]
            al = a0_v[r, pl.ds(16, 16)] + a1_v[r, pl.ds(16, 16)] + y_v[r, pl.ds(16, 16)]
            mu = jnp.maximum(d16 * am + bm, 0.0)
            lv = jnp.maximum(d16 * al + bl, 0.0)
            z = eps_v[r, pl.ds(0, 16)] * jnp.exp(0.5 * lv) + mu
            mu_v[r, pl.ds(0, 16)] = mu
            lv_v[r, pl.ds(0, 16)] = lv
            z_v[r, pl.ds(0, 16)] = z
            return carry

        lax.fori_loop(0, nrows, row, 0)
        pltpu.sync_copy(z_v.at[pl.ds(0, nrows)], z_hbm.at[pl.ds(base, nrows)])
        pltpu.sync_copy(mu_v.at[pl.ds(0, nrows)], mu_hbm.at[pl.ds(base, nrows)])
        pltpu.sync_copy(lv_v.at[pl.ds(0, nrows)], lv_hbm.at[pl.ds(base, nrows)])

    @pl.when(wid < NW - 1)
    def _():
        run(ROWS_F)

    @pl.when(wid == NW - 1)
    def _():
        run(ROWS_F_LAST)


def _finish(acc, y, dinv, b_cat, eps):
    return pl.kernel(
        _finish_body,
        out_type=(
            jax.ShapeDtypeStruct((N, 16), _f32),
            jax.ShapeDtypeStruct((N, 16), _f32),
            jax.ShapeDtypeStruct((N, 16), _f32),
        ),
        mesh=plsc.VectorSubcoreMesh(**_MESH),
        scratch_types=[
            pltpu.VMEM((ROWS_F_LAST, 32), _f32),
            pltpu.VMEM((ROWS_F_LAST, 32), _f32),
            pltpu.VMEM((ROWS_F_LAST, 32), _f32),
            pltpu.VMEM((ROWS_F_LAST,), _f32),
            pltpu.VMEM((ROWS_F_LAST, 16), _f32),
            pltpu.VMEM((32,), _f32),
            pltpu.VMEM((ROWS_F_LAST, 16), _f32),
            pltpu.VMEM((ROWS_F_LAST, 16), _f32),
            pltpu.VMEM((ROWS_F_LAST, 16), _f32),
        ],
        compiler_params=_SC_PARAMS,
    )(acc, y, dinv, b_cat, eps)


_EPS_CACHE = []


def _eps_const():
    if not _EPS_CACHE:
        try:
            with jax.ensure_compile_time_eval():
                e = jax.random.normal(jax.random.key(42), (N, 16), _f32)
            _EPS_CACHE.append(np.asarray(e))
        except Exception:
            return jax.random.normal(jax.random.key(42), (N, 16), _f32)
    return jnp.asarray(_EPS_CACHE[0])


def kernel(x, edge_index, W_mu, b_mu, W_logvar, b_logvar):
    ei3 = edge_index.reshape(2, NCH, CHUNK)
    w_cat = jnp.concatenate([W_mu, W_logvar], axis=1)
    b_cat = jnp.concatenate([b_mu, b_logvar])
    eps = _eps_const()

    xw = _mm(x, w_cat)
    partials = _deg(ei3)
    y, dinv = _scale(partials, xw)
    acc = _edges(y, ei3)
    z, mu, lv = _finish(acc, y, dinv.reshape(N), b_cat, eps)
    return (z, mu, lv)


# speedup vs baseline: 1.0558x; 1.0558x over previous
import numpy as np

import jax
import jax.numpy as jnp
from jax import lax
from jax.experimental import pallas as pl
from jax.experimental.pallas import tpu as pltpu
from jax.experimental.pallas import tpu_sc as plsc

N = 10000
NPAD = 10240
E = 320000
NW = 32
CHUNK = 128
NCH = E // CHUNK
FULL = NCH // NW
TAIL_W0 = NW - (NCH - FULL * NW)
NB = 6
LA = 3
DWIN = NPAD // 16

_f32 = jnp.float32
_SC_PARAMS = pltpu.CompilerParams(use_tc_tiling_on_sc=False, needs_layout_passes=False)
_MESH = dict(core_axis_name="c", subcore_axis_name="s", num_cores=2, num_subcores=16)


def _wid_base():
    c = lax.axis_index("c")
    s = lax.axis_index("s")
    wid = s * 2 + c
    base = wid * FULL + jnp.maximum(wid - TAIL_W0, 0)
    return wid, base


def _bcast16(vec_ref, r):
    return plsc.load_gather(vec_ref, [jnp.full((16,), 0, jnp.int32) + r])


def _mm_body(x_ref, w_ref, o_ref):
    o_ref[...] = jnp.dot(x_ref[...], w_ref[...], preferred_element_type=_f32)


def _mm(x, w_cat):
    return pl.pallas_call(
        _mm_body,
        out_shape=jax.ShapeDtypeStruct((N, 32), _f32),
    )(x, w_cat)


def _deg_body(ei_hbm, part_hbm, idx_v, deg_v):
    wid, base = _wid_base()

    def zero_row(i, carry):
        deg_v[pl.ds(i * 16, 16)] = jnp.zeros((16,), _f32)
        return carry

    lax.fori_loop(0, NPAD // 16, zero_row, 0)
    pltpu.sync_copy(ei_hbm.at[1, pl.ds(base, FULL + 1)], idx_v)

    ones = jnp.ones((16,), _f32)

    def hist_row(r):
        for k in range(CHUNK // 16):
            dvec = idx_v[r, pl.ds(k * 16, 16)]
            plsc.addupdate_scatter(deg_v, [dvec], ones)

    def row(r, carry):
        hist_row(r)
        return carry

    lax.fori_loop(0, FULL, row, 0)

    @pl.when(wid >= TAIL_W0)
    def _():
        hist_row(FULL)

    pltpu.sync_copy(deg_v, part_hbm.at[wid])


def _deg(ei3):
    return pl.kernel(
        _deg_body,
        out_type=jax.ShapeDtypeStruct((NW, NPAD), _f32),
        mesh=plsc.VectorSubcoreMesh(**_MESH),
        scratch_types=[
            pltpu.VMEM((FULL + 1, CHUNK), jnp.int32),
            pltpu.VMEM((NPAD,), _f32),
        ],
        compiler_params=_SC_PARAMS,
    )(ei3)


def _edges_body(xw_hbm, part_hbm, ei_hbm, acc_hbm, dinv_hbm,
                src_v, dst_v, rows, buf_v, pbuf_v, dv_v,
                y_sh, acc_sh, dinv_sh, sem_g, sem_s):
    c = lax.axis_index("c")
    s = lax.axis_index("s")
    wid, base = _wid_base()
    win = s * DWIN

    def zero_row(i, carry):
        buf_v[i, pl.ds(0, 16)] = jnp.zeros((16,), _f32)
        buf_v[i, pl.ds(16, 16)] = jnp.zeros((16,), _f32)
        return carry

    lax.fori_loop(0, DWIN, zero_row, 0)
    pltpu.sync_copy(buf_v, acc_sh.at[pl.ds(win, DWIN)])
    pltpu.sync_copy(ei_hbm.at[0, pl.ds(base, FULL + 1)], src_v)
    pltpu.sync_copy(ei_hbm.at[1, pl.ds(base, FULL + 1)], dst_v)

    pltpu.sync_copy(part_hbm.at[:, pl.ds(win, DWIN)], pbuf_v)

    def drow(i, carry):
        t = pbuf_v[0, pl.ds(i * 16, 16)]
        for p in range(1, NW):
            t = t + pbuf_v[p, pl.ds(i * 16, 16)]
        t = t + 1.0
        bits = plsc.bitcast(t, jnp.int32)
        yi = plsc.bitcast(jnp.int32(0x5F3759DF) - (bits >> 1), _f32)
        for _ in range(3):
            yi = yi * (1.5 - 0.5 * t * yi * yi)
        dv_v[pl.ds(i * 16, 16)] = yi
        return carry

    lax.fori_loop(0, DWIN // 16, drow, 0)
    pltpu.sync_copy(dv_v, dinv_sh.at[pl.ds(win, DWIN)])

    @pl.when(c == 0)
    def _():
        pltpu.sync_copy(dv_v, dinv_hbm.at[pl.ds(win, DWIN)])

    def stage(nrows):
        pltpu.sync_copy(xw_hbm.at[pl.ds(win, nrows)], buf_v.at[pl.ds(0, nrows)])

        def yrow(r, carry):
            d16 = _bcast16(dv_v, r)
            buf_v[r, pl.ds(0, 16)] = buf_v[r, pl.ds(0, 16)] * d16
            buf_v[r, pl.ds(16, 16)] = buf_v[r, pl.ds(16, 16)] * d16
            return carry

        lax.fori_loop(0, nrows, yrow, 0)
        pltpu.sync_copy(buf_v.at[pl.ds(0, nrows)], y_sh.at[pl.ds(win, nrows)])

    @pl.when(s < 15)
    def _():
        stage(DWIN)

    @pl.when(s == 15)
    def _():
        stage(N - 15 * DWIN)

    plsc.subcore_barrier()

    def gather(i, b):
        pltpu.async_copy(y_sh.at[src_v.at[i]], rows.at[b], sem_g.at[b])

    def gather_wait(i, b):
        pltpu.make_async_copy(y_sh.at[src_v.at[i]], rows.at[b], sem_g.at[b]).wait()

    def scat(i, b):
        pltpu.async_copy(rows.at[b], acc_sh.at[dst_v.at[i]], sem_s.at[b], add=True)

    def scat_wait(i, b):
        pltpu.make_async_copy(rows.at[b], acc_sh.at[dst_v.at[i]], sem_s.at[b]).wait()

    for b in range(LA):
        gather(b, b)

    def group(g, carry):
        gbase = g * NB
        for b in range(NB):
            i = gbase + b
            gather_wait(i, b)
            scat(i, b)
            j = i + LA
            bj = (b + LA) % NB

            @pl.when(jnp.logical_and(j >= NB, j < FULL))
            def _():
                scat_wait(j - NB, bj)

            @pl.when(j < FULL)
            def _():
                gather(j, bj)
        return carry

    lax.fori_loop(0, FULL // NB, group, 0)
    for k in range(NB):
        scat_wait(FULL - NB + k, (FULL - NB + k) % NB)

    @pl.when(wid >= TAIL_W0)
    def _():
        pltpu.sync_copy(y_sh.at[src_v.at[FULL]], rows.at[0])
        pltpu.sync_copy(rows.at[0], acc_sh.at[dst_v.at[FULL]], add=True)

    plsc.subcore_barrier()
    pltpu.sync_copy(
        acc_sh.at[pl.ds(win, DWIN)],
        acc_hbm.at[c, pl.ds(win, DWIN)],
    )


def _edges(xw, partials, ei3):
    return pl.kernel(
        _edges_body,
        out_type=(
            jax.ShapeDtypeStruct((2, NPAD, 32), _f32),
            jax.ShapeDtypeStruct((NPAD,), _f32),
        ),
        mesh=plsc.VectorSubcoreMesh(**_MESH),
        scratch_types=[
            pltpu.VMEM((FULL + 1, CHUNK), jnp.int32),
            pltpu.VMEM((FULL + 1, CHUNK), jnp.int32),
            pltpu.VMEM((NB, CHUNK, 32), _f32),
            pltpu.VMEM((DWIN, 32), _f32),
            pltpu.VMEM((NW, DWIN), _f32),
            pltpu.VMEM((DWIN,), _f32),
            pltpu.VMEM_SHARED((NPAD, 32), _f32),
            pltpu.VMEM_SHARED((NPAD, 32), _f32),
            pltpu.VMEM_SHARED((NPAD,), _f32),
            pltpu.SemaphoreType.DMA((NB,)),
            pltpu.SemaphoreType.DMA((NB,)),
        ],
        compiler_params=_SC_PARAMS,
    )(xw, partials, ei3)


ROWS_F = 312
ROWS_F_LAST = N - ROWS_F * (NW - 1)


def _finish_body(acc_hbm, xw_hbm, dinv_hbm, b_hbm, eps_hbm, z_hbm, mu_hbm,
                 lv_hbm, a0_v, a1_v, xw_v, dinv_v, eps_v, b_v, z_v, mu_v, lv_v):
    wid, _ = _wid_base()
    base = wid * ROWS_F
    pltpu.sync_copy(b_hbm, b_v)
    bm = b_v[pl.ds(0, 16)]
    bl = b_v[pl.ds(16, 16)]

    def run(nrows):
        pltpu.sync_copy(acc_hbm.at[0, pl.ds(base, nrows)], a0_v.at[pl.ds(0, nrows)])
        pltpu.sync_copy(acc_hbm.at[1, pl.ds(base, nrows)], a1_v.at[pl.ds(0, nrows)])
        pltpu.sync_copy(xw_hbm.at[pl.ds(base, nrows)], xw_v.at[pl.ds(0, nrows)])
        pltpu.sync_copy(dinv_hbm.at[pl.ds(base, nrows)], dinv_v.at[pl.ds(0, nrows)])
        pltpu.sync_copy(eps_hbm.at[pl.ds(base, nrows)], eps_v.at[pl.ds(0, nrows)])

        def row(r, carry):
            d16 = _bcast16(dinv_v, r)
            am = a0_v[r, pl.ds(0, 16)] + a1_v[r, pl.ds(0, 16)] \
                + d16 * xw_v[r, pl.ds(0, 16)]
            al = a0_v[r, pl.ds(16, 16)] + a1_v[r, pl.ds(16, 16)] \
                + d16 * xw_v[r, pl.ds(16, 16)]
            mu = jnp.maximum(d16 * am + bm, 0.0)
            lv = jnp.maximum(d16 * al + bl, 0.0)
            z = eps_v[r, pl.ds(0, 16)] * jnp.exp(0.5 * lv) + mu
            mu_v[r, pl.ds(0, 16)] = mu
            lv_v[r, pl.ds(0, 16)] = lv
            z_v[r, pl.ds(0, 16)] = z
            return carry

        lax.fori_loop(0, nrows, row, 0)
        pltpu.sync_copy(z_v.at[pl.ds(0, nrows)], z_hbm.at[pl.ds(base, nrows)])
        pltpu.sync_copy(mu_v.at[pl.ds(0, nrows)], mu_hbm.at[pl.ds(base, nrows)])
        pltpu.sync_copy(lv_v.at[pl.ds(0, nrows)], lv_hbm.at[pl.ds(base, nrows)])

    @pl.when(wid < NW - 1)
    def _():
        run(ROWS_F)

    @pl.when(wid == NW - 1)
    def _():
        run(ROWS_F_LAST)


def _finish(acc, xw, dinv, b_cat, eps):
    return pl.kernel(
        _finish_body,
        out_type=(
            jax.ShapeDtypeStruct((N, 16), _f32),
            jax.ShapeDtypeStruct((N, 16), _f32),
            jax.ShapeDtypeStruct((N, 16), _f32),
        ),
        mesh=plsc.VectorSubcoreMesh(**_MESH),
        scratch_types=[
            pltpu.VMEM((ROWS_F_LAST, 32), _f32),
            pltpu.VMEM((ROWS_F_LAST, 32), _f32),
            pltpu.VMEM((ROWS_F_LAST, 32), _f32),
            pltpu.VMEM((ROWS_F_LAST,), _f32),
            pltpu.VMEM((ROWS_F_LAST, 16), _f32),
            pltpu.VMEM((32,), _f32),
            pltpu.VMEM((ROWS_F_LAST, 16), _f32),
            pltpu.VMEM((ROWS_F_LAST, 16), _f32),
            pltpu.VMEM((ROWS_F_LAST, 16), _f32),
        ],
        compiler_params=_SC_PARAMS,
    )(acc, xw, dinv, b_cat, eps)


_EPS_CACHE = []


def _eps_const():
    if not _EPS_CACHE:
        try:
            with jax.ensure_compile_time_eval():
                e = jax.random.normal(jax.random.key(42), (N, 16), _f32)
            _EPS_CACHE.append(np.asarray(e))
        except Exception:
            return jax.random.normal(jax.random.key(42), (N, 16), _f32)
    return jnp.asarray(_EPS_CACHE[0])


def kernel(x, edge_index, W_mu, b_mu, W_logvar, b_logvar):
    ei3 = edge_index.reshape(2, NCH, CHUNK)
    w_cat = jnp.concatenate([W_mu, W_logvar], axis=1)
    b_cat = jnp.concatenate([b_mu, b_logvar])
    eps = _eps_const()

    xw = _mm(x, w_cat)
    partials = _deg(ei3)
    acc, dinv = _edges(xw, partials, ei3)
    z, mu, lv = _finish(acc, xw, dinv, b_cat, eps)
    return (z, mu, lv)
